# baseline (speedup 1.0000x reference)
import functools
import math

import jax
import jax.numpy as jnp
from jax import lax
from jax.experimental import pallas as pl


def _proj_body(x_ref, w_ref, o_ref):
    o_ref[...] = jnp.dot(x_ref[...], w_ref[...].T,
                         preferred_element_type=jnp.float32)


def _project(X, W, blk):
    N, D = X.shape
    R = W.shape[0]
    return pl.pallas_call(
        _proj_body,
        grid=(N // blk,),
        in_specs=[
            pl.BlockSpec((blk, D), lambda i: (i, 0)),
            pl.BlockSpec((R, D), lambda i: (0, 0)),
        ],
        out_specs=pl.BlockSpec((blk, R), lambda i: (i, 0)),
        out_shape=jax.ShapeDtypeStruct((N, R), jnp.float32),
    )(X, W)


def _scores_topk_body(qa_ref, ka_ref, idx_ref, val_ref, *, S, BQ, K_SEL):
    qa = qa_ref[0]
    ka = ka_ref[0]
    R = qa.shape[-1]
    scale = 1.0 / math.sqrt(R)
    s = jnp.dot(qa, ka.T, preferred_element_type=jnp.float32) * scale

    q0 = pl.program_id(1) * BQ
    qpos = q0 + lax.broadcasted_iota(jnp.int32, (BQ, S), 0)
    kpos = lax.broadcasted_iota(jnp.int32, (BQ, S), 1)
    dist = jnp.abs(qpos - kpos).astype(jnp.float32)
    s = s + jnp.exp(dist * (-1.0 / 256.0)) * 0.1

    jiota = lax.broadcasted_iota(jnp.int32, (BQ, K_SEL), 1)
    NEG = -jnp.inf

    W = 128
    C = S // W
    T = 12
    tiota = lax.broadcasted_iota(jnp.int32, (BQ, T), 1)
    off_iota = lax.broadcasted_iota(jnp.int32, (BQ, W), 1)

    pool_v_parts = []
    pool_g_parts = []
    bound_parts = []
    for c in range(C):
        blk = s[:, c * W:(c + 1) * W]

        def tbody(t, carry):
            cur, vs, gs = carry
            m = jnp.max(cur, axis=1, keepdims=True)
            o = jnp.min(jnp.where(cur == m, off_iota, W),
                        axis=1, keepdims=True)
            cur = jnp.where(off_iota == o, NEG, cur)
            sel = tiota == t
            vs = jnp.where(sel, m, vs)
            gs = jnp.where(sel, o + c * W, gs)
            return cur, vs, gs

        _, vs, gs = lax.fori_loop(
            0, T, tbody,
            (blk,
             jnp.full((BQ, T), NEG, jnp.float32),
             jnp.zeros((BQ, T), jnp.int32)))
        pool_v_parts.append(vs)
        pool_g_parts.append(gs)
        bound_parts.append(vs[:, T - 1:T])
    pool_v = jnp.concatenate(pool_v_parts, axis=1)
    pool_g = jnp.concatenate(pool_g_parts, axis=1)
    bound = jnp.concatenate(bound_parts, axis=1)

    biota = lax.broadcasted_iota(jnp.int32, (BQ, C), 1)

    def ebody(j, carry):
        pv, vals, idxs, cnt = carry
        m = jnp.max(pv, axis=1)
        g = jnp.min(jnp.where(pv == m[:, None], pool_g, S), axis=1)
        pv = jnp.where(pool_g == g[:, None], NEG, pv)
        sel = jiota == j
        vals = jnp.where(sel, m[:, None], vals)
        idxs = jnp.where(sel, g[:, None], idxs)
        cnt = cnt + (biota == lax.shift_right_logical(g, 7)[:, None]).astype(jnp.int32)
        return pv, vals, idxs, cnt

    vals0 = jnp.full((BQ, K_SEL), NEG, jnp.float32)
    idxs0 = jnp.zeros((BQ, K_SEL), jnp.int32)
    cnt0 = jnp.zeros((BQ, C), jnp.int32)
    _, vals, idxs, cnt = lax.fori_loop(0, K_SEL, ebody,
                                       (pool_v, vals0, idxs0, cnt0))

    eb = jnp.max(jnp.where(cnt >= T, bound, NEG), axis=1)
    bad = jnp.any(vals[:, K_SEL - 1] <= eb)

    @pl.when(jnp.logical_not(bad))
    def _():
        idx_ref[0] = idxs
        val_ref[0] = vals

    @pl.when(bad)
    def _():
        kiota = lax.broadcasted_iota(jnp.int32, (BQ, S), 1)

        def nbody(j, carry):
            sw, nvals, nidxs = carry
            m = jnp.max(sw, axis=1)
            idx = jnp.min(jnp.where(sw == m[:, None], kiota, S), axis=1)
            sw = jnp.where(kiota == idx[:, None], NEG, sw)
            sel = jiota == j
            nvals = jnp.where(sel, m[:, None], nvals)
            nidxs = jnp.where(sel, idx[:, None], nidxs)
            return sw, nvals, nidxs

        _, nvals, nidxs = lax.fori_loop(0, K_SEL, nbody,
                                        (s, vals0, idxs0))
        idx_ref[0] = nidxs
        val_ref[0] = nvals


def kernel(Q, K, k, W_q, W_k):
    B, S, D = Q.shape
    R = W_q.shape[0]
    K_SEL = 64
    BQ = 256 if S % 256 == 0 else S

    Qa = _project(Q.reshape(B * S, D), W_q, min(512, B * S)).reshape(B, S, R)
    Ka = _project(K.reshape(B * S, D), W_k, min(512, B * S)).reshape(B, S, R)

    body = functools.partial(_scores_topk_body, S=S, BQ=BQ, K_SEL=K_SEL)
    idxs, vals = pl.pallas_call(
        body,
        grid=(B, S // BQ),
        in_specs=[
            pl.BlockSpec((1, BQ, R), lambda b, q: (b, q, 0)),
            pl.BlockSpec((1, S, R), lambda b, q: (b, 0, 0)),
        ],
        out_specs=[
            pl.BlockSpec((1, BQ, K_SEL), lambda b, q: (b, q, 0)),
            pl.BlockSpec((1, BQ, K_SEL), lambda b, q: (b, q, 0)),
        ],
        out_shape=[
            jax.ShapeDtypeStruct((B, S, K_SEL), jnp.int32),
            jax.ShapeDtypeStruct((B, S, K_SEL), jnp.float32),
        ],
    )(Qa, Ka)
    return (idxs, vals)


# speedup vs baseline: 1.2488x; 1.2488x over previous
import functools
import math

import jax
import jax.numpy as jnp
from jax import lax
from jax.experimental import pallas as pl
from jax.experimental.pallas import tpu as pltpu


def _proj_body(x_ref, w_ref, o_ref):
    o_ref[...] = jnp.dot(x_ref[...], w_ref[...].T,
                         preferred_element_type=jnp.float32)


def _project(X, W, blk):
    N, D = X.shape
    R = W.shape[0]
    return pl.pallas_call(
        _proj_body,
        grid=(N // blk,),
        in_specs=[
            pl.BlockSpec((blk, D), lambda i: (i, 0)),
            pl.BlockSpec((R, D), lambda i: (0, 0)),
        ],
        out_specs=pl.BlockSpec((blk, R), lambda i: (i, 0)),
        out_shape=jax.ShapeDtypeStruct((N, R), jnp.float32),
    )(X, W)


def _scores_topk_body(qa_ref, ka_ref, idx_ref, val_ref, *, S, BQ, K_SEL):
    qa = qa_ref[0]
    ka = ka_ref[0]
    R = qa.shape[-1]
    scale = 1.0 / math.sqrt(R)
    s = jnp.dot(qa, ka.T, preferred_element_type=jnp.float32) * scale

    q0 = pl.program_id(1) * BQ
    qpos = q0 + lax.broadcasted_iota(jnp.int32, (BQ, S), 0)
    kpos = lax.broadcasted_iota(jnp.int32, (BQ, S), 1)
    dist = jnp.abs(qpos - kpos).astype(jnp.float32)
    s = s + jnp.exp(dist * (-1.0 / 256.0)) * 0.1

    jiota = lax.broadcasted_iota(jnp.int32, (BQ, K_SEL), 1)
    NEG = -jnp.inf

    W = 128
    C = S // W
    T = 12
    s3 = s.reshape(BQ, C, W)
    off = lax.broadcasted_iota(jnp.int32, (BQ, C, W), 2)
    base = lax.broadcasted_iota(jnp.int32, (BQ, C), 1) * W
    seg = lax.broadcasted_iota(jnp.int32, (BQ, C * T), 1) // C

    def tbody(t, carry):
        s3, pv, pg = carry
        m = jnp.max(s3, axis=2, keepdims=True)
        o = jnp.min(jnp.where(s3 == m, off, W), axis=2, keepdims=True)
        s3 = jnp.where(off == o, NEG, s3)
        mv = m[:, :, 0]
        gv = o[:, :, 0] + base
        pv = jnp.where(seg == t, jnp.tile(mv, (1, T)), pv)
        pg = jnp.where(seg == t, jnp.tile(gv, (1, T)), pg)
        return s3, pv, pg

    _, pool_v, pool_g = lax.fori_loop(
        0, T, tbody,
        (s3,
         jnp.full((BQ, C * T), NEG, jnp.float32),
         jnp.full((BQ, C * T), S, jnp.int32)))
    bound = pool_v[:, (T - 1) * C:]

    biota = lax.broadcasted_iota(jnp.int32, (BQ, C), 1)

    def ebody(j, carry):
        pv, vals, idxs, cnt = carry
        m = jnp.max(pv, axis=1)
        g = jnp.min(jnp.where(pv == m[:, None], pool_g, S), axis=1)
        pv = jnp.where(pool_g == g[:, None], NEG, pv)
        sel = jiota == j
        vals = jnp.where(sel, m[:, None], vals)
        idxs = jnp.where(sel, g[:, None], idxs)
        cnt = cnt + (biota == lax.shift_right_logical(g, 7)[:, None]).astype(jnp.int32)
        return pv, vals, idxs, cnt

    vals0 = jnp.full((BQ, K_SEL), NEG, jnp.float32)
    idxs0 = jnp.zeros((BQ, K_SEL), jnp.int32)
    cnt0 = jnp.zeros((BQ, C), jnp.int32)
    _, vals, idxs, cnt = lax.fori_loop(0, K_SEL, ebody,
                                       (pool_v, vals0, idxs0, cnt0))

    eb = jnp.max(jnp.where(cnt >= T, bound, NEG), axis=1)
    bad = jnp.any(vals[:, K_SEL - 1] <= eb)

    @pl.when(jnp.logical_not(bad))
    def _():
        idx_ref[0] = idxs
        val_ref[0] = vals

    @pl.when(bad)
    def _():
        s2 = jnp.dot(qa, ka.T, preferred_element_type=jnp.float32) * scale
        s2 = s2 + jnp.exp(dist * (-1.0 / 256.0)) * 0.1
        kiota = lax.broadcasted_iota(jnp.int32, (BQ, S), 1)

        def nbody(j, carry):
            sw, nvals, nidxs = carry
            m = jnp.max(sw, axis=1)
            idx = jnp.min(jnp.where(sw == m[:, None], kiota, S), axis=1)
            sw = jnp.where(kiota == idx[:, None], NEG, sw)
            sel = jiota == j
            nvals = jnp.where(sel, m[:, None], nvals)
            nidxs = jnp.where(sel, idx[:, None], nidxs)
            return sw, nvals, nidxs

        _, nvals, nidxs = lax.fori_loop(0, K_SEL, nbody,
                                        (s2, vals0, idxs0))
        idx_ref[0] = nidxs
        val_ref[0] = nvals


def kernel(Q, K, k, W_q, W_k):
    B, S, D = Q.shape
    R = W_q.shape[0]
    K_SEL = 64
    BQ = 128 if S % 128 == 0 else S

    Qa = _project(Q.reshape(B * S, D), W_q, min(512, B * S)).reshape(B, S, R)
    Ka = _project(K.reshape(B * S, D), W_k, min(512, B * S)).reshape(B, S, R)

    C = S // 128
    T = 12
    body = functools.partial(_scores_topk_body, S=S, BQ=BQ, K_SEL=K_SEL)
    idxs, vals = pl.pallas_call(
        body,
        grid=(B, S // BQ),
        in_specs=[
            pl.BlockSpec((1, BQ, R), lambda b, q: (b, q, 0)),
            pl.BlockSpec((1, S, R), lambda b, q: (b, 0, 0)),
        ],
        out_specs=[
            pl.BlockSpec((1, BQ, K_SEL), lambda b, q: (b, q, 0)),
            pl.BlockSpec((1, BQ, K_SEL), lambda b, q: (b, q, 0)),
        ],
        out_shape=[
            jax.ShapeDtypeStruct((B, S, K_SEL), jnp.int32),
            jax.ShapeDtypeStruct((B, S, K_SEL), jnp.float32),
        ],
    )(Qa, Ka)
    return (idxs, vals)


# speedup vs baseline: 1.4666x; 1.1744x over previous
import functools
import math

import jax
import jax.numpy as jnp
from jax import lax
from jax.experimental import pallas as pl
from jax.experimental.pallas import tpu as pltpu


def _proj_body(x_ref, w_ref, o_ref):
    o_ref[...] = jnp.dot(x_ref[...], w_ref[...].T,
                         preferred_element_type=jnp.float32)


def _project(X, W, blk):
    N, D = X.shape
    R = W.shape[0]
    return pl.pallas_call(
        _proj_body,
        grid=(N // blk,),
        in_specs=[
            pl.BlockSpec((blk, D), lambda i: (i, 0)),
            pl.BlockSpec((R, D), lambda i: (0, 0)),
        ],
        out_specs=pl.BlockSpec((blk, R), lambda i: (i, 0)),
        out_shape=jax.ShapeDtypeStruct((N, R), jnp.float32),
    )(X, W)


def _scores_topk_body(qa_ref, ka_ref, idx_ref, val_ref, *, S, BQ, K_SEL):
    qa = qa_ref[0]
    ka = ka_ref[0]
    R = qa.shape[-1]
    scale = 1.0 / math.sqrt(R)
    s = jnp.dot(qa, ka.T, preferred_element_type=jnp.float32) * scale

    q0 = pl.program_id(1) * BQ
    qpos = q0 + lax.broadcasted_iota(jnp.int32, (BQ, S), 0)
    kpos = lax.broadcasted_iota(jnp.int32, (BQ, S), 1)
    dist = jnp.abs(qpos - kpos).astype(jnp.float32)
    s = s + jnp.exp(dist * (-1.0 / 256.0)) * 0.1

    jiota = lax.broadcasted_iota(jnp.int32, (BQ, K_SEL), 1)
    NEG = -jnp.inf

    W = 128
    C = S // W
    T = 12
    s3 = s.reshape(BQ, C, W)
    off = lax.broadcasted_iota(jnp.int32, (BQ, C, W), 2)
    base = lax.broadcasted_iota(jnp.int32, (BQ, C), 1) * W
    seg = lax.broadcasted_iota(jnp.int32, (BQ, C * T), 1) // C

    def tbody(t, carry):
        s3, pv, pg = carry
        m = jnp.max(s3, axis=2, keepdims=True)
        o = jnp.min(jnp.where(s3 == m, off, W), axis=2, keepdims=True)
        s3 = jnp.where(off == o, NEG, s3)
        mv = m[:, :, 0]
        gv = o[:, :, 0] + base
        pv = jnp.where(seg == t, jnp.tile(mv, (1, T)), pv)
        pg = jnp.where(seg == t, jnp.tile(gv, (1, T)), pg)
        return s3, pv, pg

    _, pool_v, pool_g = lax.fori_loop(
        0, T, tbody,
        (s3,
         jnp.full((BQ, C * T), NEG, jnp.float32),
         jnp.full((BQ, C * T), S, jnp.int32)))
    bound = pool_v[:, (T - 1) * C:]

    biota = lax.broadcasted_iota(jnp.int32, (BQ, C), 1)

    def ebody(j, carry):
        pv, vals, idxs, cnt = carry
        m = jnp.max(pv, axis=1)
        g = jnp.min(jnp.where(pv == m[:, None], pool_g, S), axis=1)
        pv = jnp.where(pool_g == g[:, None], NEG, pv)
        sel = jiota == j
        vals = jnp.where(sel, m[:, None], vals)
        idxs = jnp.where(sel, g[:, None], idxs)
        cnt = cnt + (biota == lax.shift_right_logical(g, 7)[:, None]).astype(jnp.int32)
        return pv, vals, idxs, cnt

    vals0 = jnp.full((BQ, K_SEL), NEG, jnp.float32)
    idxs0 = jnp.zeros((BQ, K_SEL), jnp.int32)
    cnt0 = jnp.zeros((BQ, C), jnp.int32)
    _, vals, idxs, cnt = lax.fori_loop(0, K_SEL, ebody,
                                       (pool_v, vals0, idxs0, cnt0))

    eb = jnp.max(jnp.where(cnt >= T, bound, NEG), axis=1)
    bad = jnp.any(vals[:, K_SEL - 1] <= eb)

    @pl.when(jnp.logical_not(bad))
    def _():
        idx_ref[0] = idxs
        val_ref[0] = vals

    @pl.when(bad)
    def _():
        s2 = jnp.dot(qa, ka.T, preferred_element_type=jnp.float32) * scale
        s2 = s2 + jnp.exp(dist * (-1.0 / 256.0)) * 0.1
        kiota = lax.broadcasted_iota(jnp.int32, (BQ, S), 1)

        def nbody(j, carry):
            sw, nvals, nidxs = carry
            m = jnp.max(sw, axis=1)
            idx = jnp.min(jnp.where(sw == m[:, None], kiota, S), axis=1)
            sw = jnp.where(kiota == idx[:, None], NEG, sw)
            sel = jiota == j
            nvals = jnp.where(sel, m[:, None], nvals)
            nidxs = jnp.where(sel, idx[:, None], nidxs)
            return sw, nvals, nidxs

        _, nvals, nidxs = lax.fori_loop(0, K_SEL, nbody,
                                        (s2, vals0, idxs0))
        idx_ref[0] = nidxs
        val_ref[0] = nvals


def kernel(Q, K, k, W_q, W_k):
    B, S, D = Q.shape
    R = W_q.shape[0]
    K_SEL = 64
    BQ = 256 if S % 256 == 0 else S

    Qa = _project(Q.reshape(B * S, D), W_q, min(512, B * S)).reshape(B, S, R)
    Ka = _project(K.reshape(B * S, D), W_k, min(512, B * S)).reshape(B, S, R)

    C = S // 128
    T = 12
    body = functools.partial(_scores_topk_body, S=S, BQ=BQ, K_SEL=K_SEL)
    idxs, vals = pl.pallas_call(
        body,
        grid=(B, S // BQ),
        in_specs=[
            pl.BlockSpec((1, BQ, R), lambda b, q: (b, q, 0)),
            pl.BlockSpec((1, S, R), lambda b, q: (b, 0, 0)),
        ],
        out_specs=[
            pl.BlockSpec((1, BQ, K_SEL), lambda b, q: (b, q, 0)),
            pl.BlockSpec((1, BQ, K_SEL), lambda b, q: (b, q, 0)),
        ],
        out_shape=[
            jax.ShapeDtypeStruct((B, S, K_SEL), jnp.int32),
            jax.ShapeDtypeStruct((B, S, K_SEL), jnp.float32),
        ],
    )(Qa, Ka)
    return (idxs, vals)


# speedup vs baseline: 1.5882x; 1.0829x over previous
import functools
import math

import jax
import jax.numpy as jnp
from jax import lax
from jax.experimental import pallas as pl
from jax.experimental.pallas import tpu as pltpu


def _proj_body(x_ref, w_ref, o_ref):
    o_ref[...] = jnp.dot(x_ref[...], w_ref[...].T,
                         preferred_element_type=jnp.float32)


def _project(X, W, blk):
    N, D = X.shape
    R = W.shape[0]
    return pl.pallas_call(
        _proj_body,
        grid=(N // blk,),
        in_specs=[
            pl.BlockSpec((blk, D), lambda i: (i, 0)),
            pl.BlockSpec((R, D), lambda i: (0, 0)),
        ],
        out_specs=pl.BlockSpec((blk, R), lambda i: (i, 0)),
        out_shape=jax.ShapeDtypeStruct((N, R), jnp.float32),
    )(X, W)


def _scores_topk_body(qa_ref, ka_ref, idx_ref, val_ref, *, S, BQ, K_SEL):
    qa = qa_ref[0]
    ka = ka_ref[0]
    R = qa.shape[-1]
    scale = 1.0 / math.sqrt(R)
    s = jnp.dot(qa, ka.T, preferred_element_type=jnp.float32) * scale

    q0 = pl.program_id(1) * BQ
    qpos = q0 + lax.broadcasted_iota(jnp.int32, (BQ, S), 0)
    kpos = lax.broadcasted_iota(jnp.int32, (BQ, S), 1)
    dist = jnp.abs(qpos - kpos).astype(jnp.float32)
    s = s + jnp.exp(dist * (-1.0 / 256.0)) * 0.1

    jiota = lax.broadcasted_iota(jnp.int32, (BQ, K_SEL), 1)
    NEG = -jnp.inf

    W = 128
    C = S // W
    T = 12
    s3 = s.reshape(BQ, C, W)
    off = lax.broadcasted_iota(jnp.int32, (BQ, C, W), 2)
    base = lax.broadcasted_iota(jnp.int32, (BQ, C), 1) * W
    seg = lax.broadcasted_iota(jnp.int32, (BQ, C * T), 1) // C

    def tbody(t, carry):
        s3, pv, pg = carry
        m = jnp.max(s3, axis=2, keepdims=True)
        o = jnp.min(jnp.where(s3 == m, off, W), axis=2, keepdims=True)
        s3 = jnp.where(off == o, NEG, s3)
        mv = m[:, :, 0]
        gv = o[:, :, 0] + base
        pv = jnp.where(seg == t, jnp.tile(mv, (1, T)), pv)
        pg = jnp.where(seg == t, jnp.tile(gv, (1, T)), pg)
        return s3, pv, pg

    _, pool_v, pool_g = lax.fori_loop(
        0, T, tbody,
        (s3,
         jnp.full((BQ, C * T), NEG, jnp.float32),
         jnp.full((BQ, C * T), S, jnp.int32)))
    bound = pool_v[:, (T - 1) * C:]

    biota = lax.broadcasted_iota(jnp.int32, (BQ, C), 1)

    def ebody(j, carry):
        pv, vals, idxs, cnt = carry
        m = jnp.max(pv, axis=1)
        g = jnp.min(jnp.where(pv == m[:, None], pool_g, S), axis=1)
        pv = jnp.where(pool_g == g[:, None], NEG, pv)
        sel = jiota == j
        vals = jnp.where(sel, m[:, None], vals)
        idxs = jnp.where(sel, g[:, None], idxs)
        cnt = cnt + (biota == lax.shift_right_logical(g, 7)[:, None]).astype(jnp.int32)
        return pv, vals, idxs, cnt

    vals0 = jnp.full((BQ, K_SEL), NEG, jnp.float32)
    idxs0 = jnp.zeros((BQ, K_SEL), jnp.int32)
    cnt0 = jnp.zeros((BQ, C), jnp.int32)
    _, vals, idxs, cnt = lax.fori_loop(0, K_SEL, ebody,
                                       (pool_v, vals0, idxs0, cnt0))

    eb = jnp.max(jnp.where(cnt >= T, bound, NEG), axis=1)
    bad = jnp.any(vals[:, K_SEL - 1] <= eb)

    @pl.when(jnp.logical_not(bad))
    def _():
        idx_ref[0] = idxs
        val_ref[0] = vals

    @pl.when(bad)
    def _():
        s2 = jnp.dot(qa, ka.T, preferred_element_type=jnp.float32) * scale
        s2 = s2 + jnp.exp(dist * (-1.0 / 256.0)) * 0.1
        kiota = lax.broadcasted_iota(jnp.int32, (BQ, S), 1)

        def nbody(j, carry):
            sw, nvals, nidxs = carry
            m = jnp.max(sw, axis=1)
            idx = jnp.min(jnp.where(sw == m[:, None], kiota, S), axis=1)
            sw = jnp.where(kiota == idx[:, None], NEG, sw)
            sel = jiota == j
            nvals = jnp.where(sel, m[:, None], nvals)
            nidxs = jnp.where(sel, idx[:, None], nidxs)
            return sw, nvals, nidxs

        _, nvals, nidxs = lax.fori_loop(0, K_SEL, nbody,
                                        (s2, vals0, idxs0))
        idx_ref[0] = nidxs
        val_ref[0] = nvals


def kernel(Q, K, k, W_q, W_k):
    B, S, D = Q.shape
    R = W_q.shape[0]
    K_SEL = 64
    BQ = 512 if S % 512 == 0 else S

    Qa = _project(Q.reshape(B * S, D), W_q, min(512, B * S)).reshape(B, S, R)
    Ka = _project(K.reshape(B * S, D), W_k, min(512, B * S)).reshape(B, S, R)

    C = S // 128
    T = 12
    body = functools.partial(_scores_topk_body, S=S, BQ=BQ, K_SEL=K_SEL)
    idxs, vals = pl.pallas_call(
        body,
        grid=(B, S // BQ),
        in_specs=[
            pl.BlockSpec((1, BQ, R), lambda b, q: (b, q, 0)),
            pl.BlockSpec((1, S, R), lambda b, q: (b, 0, 0)),
        ],
        out_specs=[
            pl.BlockSpec((1, BQ, K_SEL), lambda b, q: (b, q, 0)),
            pl.BlockSpec((1, BQ, K_SEL), lambda b, q: (b, q, 0)),
        ],
        out_shape=[
            jax.ShapeDtypeStruct((B, S, K_SEL), jnp.int32),
            jax.ShapeDtypeStruct((B, S, K_SEL), jnp.float32),
        ],
    )(Qa, Ka)
    return (idxs, vals)
